# baseline (speedup 1.0000x reference)
import functools

import jax
import jax.numpy as jnp
from jax import lax
from jax.experimental import pallas as pl
from jax.experimental.pallas import tpu as pltpu
from jax.experimental.pallas import tpu_sc as plsc

B, E, N = 64, 1024, 128
NC, NS, L = 2, 16, 16
NW = NC * NS
BPW = B // NW
G = 32


def _tc_q_body(p_ref, d_ref, q_ref):
    d = d_ref[...]
    for g in range(G):
        p = p_ref[g]
        m = jnp.dot(p, d, preferred_element_type=jnp.float32)
        q_ref[g] = lax.dot_general(m, p, (((1,), (1,)), ((), ())),
                                   preferred_element_type=jnp.float32)


def _compute_q(P, d_error):
    return pl.pallas_call(
        _tc_q_body,
        grid=(B // G,),
        in_specs=[
            pl.BlockSpec((G, N, N), lambda b: (b, 0, 0)),
            pl.BlockSpec((N, N), lambda b: (0, 0)),
        ],
        out_specs=pl.BlockSpec((G, N, N), lambda b: (b, 0, 0)),
        out_shape=jax.ShapeDtypeStruct((B, N, N), jnp.float32),
    )(P, d_error)


@functools.partial(
    pl.kernel,
    out_type=jax.ShapeDtypeStruct((B, L), jnp.float32),
    mesh=plsc.VectorSubcoreMesh(core_axis_name="c", subcore_axis_name="s",
                                num_cores=NC, num_subcores=NS),
    compiler_params=pltpu.CompilerParams(needs_layout_passes=False),
    scratch_types=[
        pltpu.VMEM((BPW, N, N), jnp.float32),
        pltpu.VMEM((BPW, E), jnp.int32),
        pltpu.VMEM((BPW, E), jnp.int32),
        pltpu.VMEM((BPW, E), jnp.float32),
        pltpu.VMEM((BPW, L), jnp.float32),
        pltpu.SemaphoreType.DMA,
    ],
)
def _sc_edge_reduce(q_hbm, i_hbm, j_hbm, w_hbm, out_hbm,
                    q_v, i_v, j_v, w_v, out_v, sem):
    wid = lax.axis_index("s") * NC + lax.axis_index("c")
    b0 = wid * BPW
    cps = [
        pltpu.async_copy(q_hbm.at[pl.ds(b0, BPW)], q_v, sem),
        pltpu.async_copy(i_hbm.at[pl.ds(b0, BPW)], i_v, sem),
        pltpu.async_copy(j_hbm.at[pl.ds(b0, BPW)], j_v, sem),
        pltpu.async_copy(w_hbm.at[pl.ds(b0, BPW)], w_v, sem),
    ]
    for cp in cps:
        cp.wait()
    lanes = lax.iota(jnp.int32, L)
    for local in range(BPW):
        lc = jnp.full((L,), local, jnp.int32)

        def body(k, carry):
            acc, wsum = carry
            off = lanes + k * L
            iv = plsc.load_gather(i_v, [lc, off])
            jv = plsc.load_gather(j_v, [lc, off])
            vals = plsc.load_gather(q_v, [lc, iv, jv])
            wk = plsc.load_gather(w_v, [lc, off])
            return acc + wk * vals, wsum + wk

        acc, wsum = lax.fori_loop(
            0, E // L, body,
            (jnp.zeros((L,), jnp.float32), jnp.zeros((L,), jnp.float32)))
        svec = jnp.full((L,), jnp.sum(acc), jnp.float32)
        wvec = jnp.full((L,), jnp.maximum(jnp.sum(wsum), 1e-8), jnp.float32)
        plsc.store_scatter(out_v, [lc, lanes], svec / wvec)
    pltpu.async_copy(out_v, out_hbm.at[pl.ds(b0, BPW)], sem).wait()


def kernel(P, d_error, circuit_edge_pairs, circuit_edge_weights):
    Q = _compute_q(P, d_error)
    i_idx = circuit_edge_pairs[..., 0]
    j_idx = circuit_edge_pairs[..., 1]
    per_sample = _sc_edge_reduce(Q, i_idx, j_idx, circuit_edge_weights)
    return jnp.sum(per_sample[:, 0]) / B


# speedup vs baseline: 1.0086x; 1.0086x over previous
import functools

import jax
import jax.numpy as jnp
from jax import lax
from jax.experimental import pallas as pl
from jax.experimental.pallas import tpu as pltpu
from jax.experimental.pallas import tpu_sc as plsc

B, E, N = 64, 1024, 128
NC, NS, L = 2, 16, 16
NW = NC * NS
BPW = B // NW
G = 16


def _tc_q_body(p_ref, d_ref, q_ref):
    d = d_ref[...]
    for g in range(G):
        p = p_ref[g]
        m = jnp.dot(p, d, preferred_element_type=jnp.float32)
        q_ref[g] = lax.dot_general(m, p, (((1,), (1,)), ((), ())),
                                   preferred_element_type=jnp.float32)


def _compute_q(P, d_error):
    return pl.pallas_call(
        _tc_q_body,
        grid=(B // G,),
        in_specs=[
            pl.BlockSpec((G, N, N), lambda b: (b, 0, 0)),
            pl.BlockSpec((N, N), lambda b: (0, 0)),
        ],
        out_specs=pl.BlockSpec((G, N, N), lambda b: (b, 0, 0)),
        out_shape=jax.ShapeDtypeStruct((B, N, N), jnp.float32),
    )(P, d_error)


@functools.partial(
    pl.kernel,
    out_type=jax.ShapeDtypeStruct((B, L), jnp.float32),
    mesh=plsc.VectorSubcoreMesh(core_axis_name="c", subcore_axis_name="s",
                                num_cores=NC, num_subcores=NS),
    compiler_params=pltpu.CompilerParams(needs_layout_passes=False),
    scratch_types=[
        pltpu.VMEM((BPW, N, N), jnp.float32),
        pltpu.VMEM((BPW, 2 * E), jnp.int32),
        pltpu.VMEM((BPW, E), jnp.float32),
        pltpu.VMEM((BPW, L), jnp.float32),
        pltpu.SemaphoreType.DMA,
    ],
)
def _sc_edge_reduce(q_hbm, pairs_hbm, w_hbm, out_hbm,
                    q_v, pairs_v, w_v, out_v, sem):
    wid = lax.axis_index("s") * NC + lax.axis_index("c")
    b0 = wid * BPW
    cps = [
        pltpu.async_copy(q_hbm.at[pl.ds(b0, BPW)], q_v, sem),
        pltpu.async_copy(pairs_hbm.at[pl.ds(b0, BPW)], pairs_v, sem),
        pltpu.async_copy(w_hbm.at[pl.ds(b0, BPW)], w_v, sem),
    ]
    for cp in cps:
        cp.wait()
    lanes = lax.iota(jnp.int32, L)
    for local in range(BPW):
        lc = jnp.full((L,), local, jnp.int32)

        def body(k, carry):
            acc0, acc1, ws0, ws1 = carry
            off0 = lanes + k * (2 * L)
            off1 = off0 + L
            iv0 = plsc.load_gather(pairs_v, [lc, off0 * 2])
            jv0 = plsc.load_gather(pairs_v, [lc, off0 * 2 + 1])
            iv1 = plsc.load_gather(pairs_v, [lc, off1 * 2])
            jv1 = plsc.load_gather(pairs_v, [lc, off1 * 2 + 1])
            v0 = plsc.load_gather(q_v, [lc, iv0, jv0])
            v1 = plsc.load_gather(q_v, [lc, iv1, jv1])
            wk0 = plsc.load_gather(w_v, [lc, off0])
            wk1 = plsc.load_gather(w_v, [lc, off1])
            return (acc0 + wk0 * v0, acc1 + wk1 * v1,
                    ws0 + wk0, ws1 + wk1)

        z = jnp.zeros((L,), jnp.float32)
        acc0, acc1, ws0, ws1 = lax.fori_loop(
            0, E // (2 * L), body, (z, z, z, z))
        acc = acc0 + acc1
        wsum = ws0 + ws1
        svec = jnp.full((L,), jnp.sum(acc), jnp.float32)
        wvec = jnp.full((L,), jnp.maximum(jnp.sum(wsum), 1e-8), jnp.float32)
        plsc.store_scatter(out_v, [lc, lanes], svec / wvec)
    pltpu.async_copy(out_v, out_hbm.at[pl.ds(b0, BPW)], sem).wait()


def kernel(P, d_error, circuit_edge_pairs, circuit_edge_weights):
    Q = _compute_q(P, d_error)
    pairs_flat = circuit_edge_pairs.reshape(B, 2 * E)
    per_sample = _sc_edge_reduce(Q, pairs_flat, circuit_edge_weights)
    return jnp.sum(per_sample[:, 0]) / B
